# baseline (speedup 1.0000x reference)
import functools

import jax
import jax.numpy as jnp
from jax import lax
from jax.experimental import pallas as pl
from jax.experimental.pallas import tpu as pltpu
from jax.experimental.pallas import tpu_sc as plsc

T = 2048
H = 768
E = 8
K = 2
MI = 1024
ISH = 2048

BT = 256
NBLK = T * K // BT + E
NBLK_PAD = 32
PADN = NBLK * BT

NC, NS, NW, L = 2, 16, 32, 16


def _router_body(x_ref, wg_ref, logits_ref, topw_ref, slots_ref, eid_ref):
    x = x_ref[...]
    wg = wg_ref[...]
    logits = lax.dot_general(x, wg, (((1,), (1,)), ((), ())),
                             preferred_element_type=jnp.float32)
    logits_ref[...] = logits
    m = jnp.max(logits, axis=1, keepdims=True)
    ex = jnp.exp(logits - m)
    rw = ex / jnp.sum(ex, axis=1, keepdims=True)
    iota_e = lax.broadcasted_iota(jnp.int32, (T, E), 1)
    m0 = jnp.max(rw, axis=1, keepdims=True)
    i0 = jnp.min(jnp.where(rw == m0, iota_e, E), axis=1, keepdims=True)
    rw1 = jnp.where(iota_e == i0, -1.0, rw)
    m1 = jnp.max(rw1, axis=1, keepdims=True)
    i1 = jnp.min(jnp.where(rw1 == m1, iota_e, E), axis=1, keepdims=True)
    topw_ref[...] = jnp.concatenate([m0, m1], axis=1)
    oh0 = (iota_e == i0).astype(jnp.float32)
    oh1 = (iota_e == i1).astype(jnp.float32)
    cnt = oh0 + oh1
    s = cnt
    sh = 1
    while sh < T:
        s = s + jnp.concatenate(
            [jnp.zeros((sh, E), jnp.float32), s[:-sh, :]], axis=0)
        sh *= 2
    pre = s - cnt
    tot = s[T - 1:T, :]
    nb = jnp.floor((tot + (BT - 1)) / BT)
    ends = nb
    sh = 1
    while sh < E:
        ends = ends + jnp.concatenate(
            [jnp.zeros((1, sh), jnp.float32), ends[:, :-sh]], axis=1)
        sh *= 2
    offrow = (ends - nb) * float(BT)
    slot0 = jnp.sum(oh0 * (offrow + pre), axis=1, keepdims=True)
    slot1 = jnp.sum(oh1 * (offrow + pre), axis=1, keepdims=True)
    slots_ref[...] = jnp.concatenate([slot0, slot1], axis=1).astype(jnp.int32)
    bio = lax.broadcasted_iota(jnp.int32, (NBLK_PAD, E), 0).astype(jnp.float32)
    ge = (bio >= jnp.broadcast_to(ends, (NBLK_PAD, E))).astype(jnp.float32)
    eidf = jnp.minimum(jnp.sum(ge, axis=1, keepdims=True), float(E - 1))
    eid_ref[...] = eidf.astype(jnp.int32)


_router = pl.pallas_call(
    _router_body,
    out_shape=[
        jax.ShapeDtypeStruct((T, E), jnp.float32),
        jax.ShapeDtypeStruct((T, K), jnp.float32),
        jax.ShapeDtypeStruct((T, K), jnp.int32),
        jax.ShapeDtypeStruct((NBLK_PAD, 1), jnp.int32),
    ],
)


def _sc_scatter_body(p0_hbm, p1_hbm, w0_hbm, w1_hbm, src_hbm, wsl_hbm,
                     p0_v, p1_v, w0_v, w1_v, src_v, wsl_v):
    wid = lax.axis_index("s") * NC + lax.axis_index("c")

    @pl.when(wid == 0)
    def _():
        pltpu.sync_copy(p0_hbm, p0_v)
        pltpu.sync_copy(p1_hbm, p1_v)
        pltpu.sync_copy(w0_hbm, w0_v)
        pltpu.sync_copy(w1_hbm, w1_v)

        @pl.loop(0, PADN // L)
        def _(i):
            src_v[pl.ds(i * L, L)] = jnp.zeros((L,), jnp.int32)
            wsl_v[pl.ds(i * L, L)] = jnp.zeros((L,), jnp.float32)

        @pl.loop(0, T // L)
        def _(i):
            base = i * L
            tvec = lax.iota(jnp.int32, L) + base
            idx0 = p0_v[pl.ds(base, L)]
            idx1 = p1_v[pl.ds(base, L)]
            plsc.store_scatter(src_v, [idx0], tvec)
            plsc.store_scatter(wsl_v, [idx0], w0_v[pl.ds(base, L)])
            plsc.store_scatter(src_v, [idx1], tvec)
            plsc.store_scatter(wsl_v, [idx1], w1_v[pl.ds(base, L)])

        pltpu.sync_copy(src_v, src_hbm)
        pltpu.sync_copy(wsl_v, wsl_hbm)


RPW = PADN // NW
RCH = 96
H2 = H // 2


def _sc_gather_body(x_hbm, src_hbm, xs_hbm, idx_v, ba, bb, ga, gb, wa, wb):
    sid = lax.axis_index("s")
    wid = sid * NC + lax.axis_index("c")
    base = wid * RPW
    pltpu.sync_copy(src_hbm.at[pl.ds(base, RPW)], idx_v)
    bufs = (ba, bb)
    gsems = (ga, gb)
    wsems = (wa, wb)
    nr = RPW // RCH
    gathers = [None] * nr
    writes = [None] * nr
    for r in range(nr):
        idx_sl = idx_v.at[pl.ds(r * RCH, RCH)]
        gathers[r] = pltpu.async_copy(x_hbm.at[idx_sl], bufs[r % 2],
                                      gsems[r % 2])
    for r in range(nr):
        gathers[r].wait()
        writes[r] = pltpu.async_copy(
            bufs[r % 2], xs_hbm.at[pl.ds(base + r * RCH, RCH)],
            wsems[r % 2])
    for r in range(nr):
        writes[r].wait()


def _moe_body(eid_ref, xs_ref, weg_ref, weu_ref, wed_ref, wsl_ref, out_ref):
    del eid_ref
    xb = xs_ref[...]
    wg = weg_ref[0]
    wu = weu_ref[0]
    wd = wed_ref[0]
    g = lax.dot_general(xb, wg, (((1,), (1,)), ((), ())),
                        preferred_element_type=jnp.float32)
    u = lax.dot_general(xb, wu, (((1,), (1,)), ((), ())),
                        preferred_element_type=jnp.float32)
    h = ((g * jax.nn.sigmoid(g)) * u).astype(jnp.bfloat16)
    o = lax.dot_general(h, wd, (((1,), (1,)), ((), ())),
                        preferred_element_type=jnp.float32)
    out_ref[...] = o * wsl_ref[0]


_moe_mm = pl.pallas_call(
    _moe_body,
    grid_spec=pltpu.PrefetchScalarGridSpec(
        num_scalar_prefetch=1,
        grid=(NBLK,),
        in_specs=[
            pl.BlockSpec((BT, H), lambda b, eid: (b, 0)),
            pl.BlockSpec((1, MI, H), lambda b, eid: (eid[b], 0, 0)),
            pl.BlockSpec((1, MI, H), lambda b, eid: (eid[b], 0, 0)),
            pl.BlockSpec((1, H, MI), lambda b, eid: (eid[b], 0, 0)),
            pl.BlockSpec((1, BT, 1), lambda b, eid: (b, 0, 0)),
        ],
        out_specs=pl.BlockSpec((BT, H), lambda b, eid: (b, 0)),
    ),
    out_shape=jax.ShapeDtypeStruct((PADN, H), jnp.float32),
)


TPW = T // NW


CCH = TPW // 2


def _sc_combine_body(eo_hbm, p0_hbm, p1_hbm, moe_hbm,
                     i0_v, i1_v, r0a, r0b, r1a, r1b,
                     s0a, s0b, s1a, s1b, sw):
    wid = lax.axis_index("s") * NC + lax.axis_index("c")
    base = wid * TPW
    pltpu.sync_copy(p0_hbm.at[pl.ds(base, TPW)], i0_v)
    pltpu.sync_copy(p1_hbm.at[pl.ds(base, TPW)], i1_v)
    c = [
        pltpu.async_copy(eo_hbm.at[i0_v.at[pl.ds(0, CCH)]], r0a, s0a),
        pltpu.async_copy(eo_hbm.at[i0_v.at[pl.ds(CCH, CCH)]], r0b, s0b),
        pltpu.async_copy(eo_hbm.at[i1_v.at[pl.ds(0, CCH)]], r1a, s1a),
        pltpu.async_copy(eo_hbm.at[i1_v.at[pl.ds(CCH, CCH)]], r1b, s1b),
    ]
    writes = []
    for half, (ra, rb) in enumerate(((r0a, r1a), (r0b, r1b))):
        c[half].wait()
        c[half + 2].wait()

        @pl.loop(0, CCH)
        def _(j):
            @pl.loop(0, H // L, unroll=8)
            def _(cchunk):
                sl = pl.ds(cchunk * L, L)
                ra[j, sl] = ra[j, sl] + rb[j, sl]

        writes.append(pltpu.async_copy(
            ra, moe_hbm.at[pl.ds(base + half * CCH, CCH)], sw))
    for w in writes:
        w.wait()


@functools.lru_cache(maxsize=1)
def _build_sc_kernels():
    mesh = plsc.VectorSubcoreMesh(core_axis_name="c", subcore_axis_name="s")
    sc_scatter = functools.partial(
        pl.kernel,
        out_type=[jax.ShapeDtypeStruct((PADN,), jnp.int32),
                  jax.ShapeDtypeStruct((PADN,), jnp.float32)],
        mesh=mesh,
        scratch_types=[
            pltpu.VMEM((T,), jnp.int32), pltpu.VMEM((T,), jnp.int32),
            pltpu.VMEM((T,), jnp.float32), pltpu.VMEM((T,), jnp.float32),
            pltpu.VMEM((PADN,), jnp.int32), pltpu.VMEM((PADN,), jnp.float32),
        ],
        compiler_params=pltpu.CompilerParams(needs_layout_passes=False),
    )(_sc_scatter_body)
    sc_gather = functools.partial(
        pl.kernel,
        out_type=jax.ShapeDtypeStruct((PADN, H2), jnp.int32),
        mesh=mesh,
        scratch_types=[pltpu.VMEM((RPW,), jnp.int32),
                       pltpu.VMEM((RCH, H2), jnp.int32),
                       pltpu.VMEM((RCH, H2), jnp.int32),
                       pltpu.SemaphoreType.DMA, pltpu.SemaphoreType.DMA,
                       pltpu.SemaphoreType.DMA, pltpu.SemaphoreType.DMA],
    )(_sc_gather_body)
    sc_combine = functools.partial(
        pl.kernel,
        out_type=jax.ShapeDtypeStruct((T, H), jnp.float32),
        mesh=mesh,
        scratch_types=[pltpu.VMEM((TPW,), jnp.int32),
                       pltpu.VMEM((TPW,), jnp.int32),
                       pltpu.VMEM((CCH, H), jnp.float32),
                       pltpu.VMEM((CCH, H), jnp.float32),
                       pltpu.VMEM((CCH, H), jnp.float32),
                       pltpu.VMEM((CCH, H), jnp.float32),
                       pltpu.SemaphoreType.DMA, pltpu.SemaphoreType.DMA,
                       pltpu.SemaphoreType.DMA, pltpu.SemaphoreType.DMA,
                       pltpu.SemaphoreType.DMA],
    )(_sc_combine_body)
    return sc_scatter, sc_gather, sc_combine


BTF = 256


def _shared_body(x_ref, wsg_ref, wsu_ref, wsd_ref, wsig_ref, out_ref):
    xb = x_ref[...]
    g = lax.dot_general(xb, wsg_ref[...], (((1,), (1,)), ((), ())),
                        preferred_element_type=jnp.float32)
    u = lax.dot_general(xb, wsu_ref[...], (((1,), (1,)), ((), ())),
                        preferred_element_type=jnp.float32)
    h = ((g * jax.nn.sigmoid(g)) * u).astype(jnp.bfloat16)
    shd = lax.dot_general(h, wsd_ref[...], (((1,), (1,)), ((), ())),
                          preferred_element_type=jnp.float32)
    sg = jax.nn.sigmoid(jnp.sum(
        xb.astype(jnp.float32) * wsig_ref[...].astype(jnp.float32),
        axis=1, keepdims=True))
    out_ref[...] = sg * shd


_shared = pl.pallas_call(
    _shared_body,
    grid=(T // BTF,),
    in_specs=[
        pl.BlockSpec((BTF, H), lambda b: (b, 0)),
        pl.BlockSpec((ISH, H), lambda b: (0, 0)),
        pl.BlockSpec((ISH, H), lambda b: (0, 0)),
        pl.BlockSpec((H, ISH), lambda b: (0, 0)),
        pl.BlockSpec((1, H), lambda b: (0, 0)),
    ],
    out_specs=pl.BlockSpec((BTF, H), lambda b: (b, 0)),
    out_shape=jax.ShapeDtypeStruct((T, H), jnp.float32),
)


def _fadd_body(moe_ref, sgsh_ref, out_ref):
    out_ref[...] = moe_ref[...] + sgsh_ref[...]


_fadd = pl.pallas_call(
    _fadd_body,
    grid=(T // 512,),
    in_specs=[
        pl.BlockSpec((512, H), lambda b: (b, 0)),
        pl.BlockSpec((512, H), lambda b: (b, 0)),
    ],
    out_specs=pl.BlockSpec((512, H), lambda b: (b, 0)),
    out_shape=jax.ShapeDtypeStruct((T, H), jnp.float32),
)


def kernel(hidden_states, Wg, We_gate, We_up, We_down,
           Ws_gate, Ws_up, Ws_down, Wsg):
    b, s_, h = hidden_states.shape
    x = hidden_states.reshape(s_, h)
    logits, topw, slots, eid2 = _router(x, Wg)
    p0 = slots[:, 0]
    p1 = slots[:, 1]
    w0 = topw[:, 0]
    w1 = topw[:, 1]
    eid_arr = eid2.reshape(NBLK_PAD)[:NBLK]
    bf = jnp.bfloat16
    x16 = x.astype(bf)
    x16i = lax.bitcast_convert_type(x16.reshape(T, H2, 2), jnp.int32)
    _sc_scatter, _sc_gather, _sc_combine = _build_sc_kernels()
    src_tok, wslot = _sc_scatter(p0, p1, w0, w1)
    sgsh = _shared(x16, Ws_gate.astype(bf), Ws_up.astype(bf),
                   Ws_down.astype(bf), Wsg.astype(bf))
    xs32 = _sc_gather(x16i, src_tok)
    xs = lax.bitcast_convert_type(xs32, bf).reshape(PADN, H)
    eo = _moe_mm(eid_arr, xs, We_gate.astype(bf), We_up.astype(bf),
                 We_down.astype(bf), wslot.reshape(NBLK, BT, 1))
    moe = _sc_combine(eo, p0, p1)
    final = _fadd(moe, sgsh)
    return final.reshape(b, s_, h), logits


# speedup vs baseline: 1.6382x; 1.6382x over previous
import functools

import jax
import jax.numpy as jnp
from jax import lax
from jax.experimental import pallas as pl
from jax.experimental.pallas import tpu as pltpu
from jax.experimental.pallas import tpu_sc as plsc

T = 2048
H = 768
E = 8
K = 2
MI = 1024
ISH = 2048

BT = 256
NBLK = T * K // BT + E
NBLK_PAD = 32
PADN = NBLK * BT

NC, NS, NW, L = 2, 16, 32, 16


def _router_body(x_ref, wg_ref, logits_ref, topw_ref, slots_ref, eid_ref):
    x = x_ref[...]
    wg = wg_ref[...]
    logits = lax.dot_general(x, wg, (((1,), (1,)), ((), ())),
                             preferred_element_type=jnp.float32)
    logits_ref[...] = logits
    m = jnp.max(logits, axis=1, keepdims=True)
    ex = jnp.exp(logits - m)
    rw = ex / jnp.sum(ex, axis=1, keepdims=True)
    iota_e = lax.broadcasted_iota(jnp.int32, (T, E), 1)
    m0 = jnp.max(rw, axis=1, keepdims=True)
    i0 = jnp.min(jnp.where(rw == m0, iota_e, E), axis=1, keepdims=True)
    rw1 = jnp.where(iota_e == i0, -1.0, rw)
    m1 = jnp.max(rw1, axis=1, keepdims=True)
    i1 = jnp.min(jnp.where(rw1 == m1, iota_e, E), axis=1, keepdims=True)
    topw_ref[...] = jnp.concatenate([m0, m1], axis=1)
    oh0 = (iota_e == i0).astype(jnp.float32)
    oh1 = (iota_e == i1).astype(jnp.float32)
    cnt = oh0 + oh1
    s = cnt
    sh = 1
    while sh < T:
        s = s + jnp.concatenate(
            [jnp.zeros((sh, E), jnp.float32), s[:-sh, :]], axis=0)
        sh *= 2
    pre = s - cnt
    tot = s[T - 1:T, :]
    nb = jnp.floor((tot + (BT - 1)) / BT)
    ends = nb
    sh = 1
    while sh < E:
        ends = ends + jnp.concatenate(
            [jnp.zeros((1, sh), jnp.float32), ends[:, :-sh]], axis=1)
        sh *= 2
    offrow = (ends - nb) * float(BT)
    slot0 = jnp.sum(oh0 * (offrow + pre), axis=1, keepdims=True)
    slot1 = jnp.sum(oh1 * (offrow + pre), axis=1, keepdims=True)
    slots_ref[...] = jnp.concatenate([slot0, slot1], axis=1).astype(jnp.int32)
    bio = lax.broadcasted_iota(jnp.int32, (NBLK_PAD, E), 0).astype(jnp.float32)
    ge = (bio >= jnp.broadcast_to(ends, (NBLK_PAD, E))).astype(jnp.float32)
    eidf = jnp.minimum(jnp.sum(ge, axis=1, keepdims=True), float(E - 1))
    eid_ref[...] = eidf.astype(jnp.int32)


_router = pl.pallas_call(
    _router_body,
    out_shape=[
        jax.ShapeDtypeStruct((T, E), jnp.float32),
        jax.ShapeDtypeStruct((T, K), jnp.float32),
        jax.ShapeDtypeStruct((T, K), jnp.int32),
        jax.ShapeDtypeStruct((NBLK_PAD, 1), jnp.int32),
    ],
)


def _sc_scatter_body(p0_hbm, p1_hbm, w0_hbm, w1_hbm, src_hbm, wsl_hbm,
                     p0_v, p1_v, w0_v, w1_v, src_v, wsl_v):
    wid = lax.axis_index("s") * NC + lax.axis_index("c")

    @pl.when(wid == 0)
    def _():
        pltpu.sync_copy(p0_hbm, p0_v)
        pltpu.sync_copy(p1_hbm, p1_v)
        pltpu.sync_copy(w0_hbm, w0_v)
        pltpu.sync_copy(w1_hbm, w1_v)

        @pl.loop(0, PADN // L)
        def _(i):
            src_v[pl.ds(i * L, L)] = jnp.zeros((L,), jnp.int32)
            wsl_v[pl.ds(i * L, L)] = jnp.zeros((L,), jnp.float32)

        @pl.loop(0, T // L)
        def _(i):
            base = i * L
            tvec = lax.iota(jnp.int32, L) + base
            idx0 = p0_v[pl.ds(base, L)]
            idx1 = p1_v[pl.ds(base, L)]
            plsc.store_scatter(src_v, [idx0], tvec)
            plsc.store_scatter(wsl_v, [idx0], w0_v[pl.ds(base, L)])
            plsc.store_scatter(src_v, [idx1], tvec)
            plsc.store_scatter(wsl_v, [idx1], w1_v[pl.ds(base, L)])

        pltpu.sync_copy(src_v, src_hbm)
        pltpu.sync_copy(wsl_v, wsl_hbm)


RPW = PADN // NW
RCH = 96
H2 = H // 2


def _sc_gather_body(x_hbm, src_hbm, xs_hbm, idx_v, ba, bb, ga, gb, wa, wb):
    sid = lax.axis_index("s")
    wid = sid * NC + lax.axis_index("c")
    base = wid * RPW
    pltpu.sync_copy(src_hbm.at[pl.ds(base, RPW)], idx_v)
    bufs = (ba, bb)
    gsems = (ga, gb)
    wsems = (wa, wb)
    nr = RPW // RCH
    gathers = [None] * nr
    writes = [None] * nr
    for r in range(nr):
        idx_sl = idx_v.at[pl.ds(r * RCH, RCH)]
        gathers[r] = pltpu.async_copy(x_hbm.at[idx_sl], bufs[r % 2],
                                      gsems[r % 2])
    for r in range(nr):
        gathers[r].wait()
        writes[r] = pltpu.async_copy(
            bufs[r % 2], xs_hbm.at[pl.ds(base + r * RCH, RCH)],
            wsems[r % 2])
    for r in range(nr):
        writes[r].wait()


def _moe_body(eid_ref, xs_ref, weg_ref, weu_ref, wed_ref, wsl_ref, out_ref):
    del eid_ref
    xp = xs_ref[...]
    lo = lax.bitcast_convert_type(xp << 16, jnp.float32)
    hi = lax.bitcast_convert_type(
        xp & jnp.int32(-65536), jnp.float32)
    xb = jnp.concatenate([lo, hi], axis=1).astype(jnp.bfloat16)
    wg = weg_ref[0].astype(jnp.bfloat16)
    wu = weu_ref[0].astype(jnp.bfloat16)
    wd = wed_ref[0].astype(jnp.bfloat16)
    g = lax.dot_general(xb, wg, (((1,), (1,)), ((), ())),
                        preferred_element_type=jnp.float32)
    u = lax.dot_general(xb, wu, (((1,), (1,)), ((), ())),
                        preferred_element_type=jnp.float32)
    h = ((g * jax.nn.sigmoid(g)) * u).astype(jnp.bfloat16)
    o = lax.dot_general(h, wd, (((1,), (1,)), ((), ())),
                        preferred_element_type=jnp.float32)
    out_ref[...] = o * wsl_ref[0]


_moe_mm = pl.pallas_call(
    _moe_body,
    grid_spec=pltpu.PrefetchScalarGridSpec(
        num_scalar_prefetch=1,
        grid=(NBLK,),
        in_specs=[
            pl.BlockSpec((BT, H2), lambda b, eid: (b, 0)),
            pl.BlockSpec((1, MI, H), lambda b, eid: (eid[b], 0, 0)),
            pl.BlockSpec((1, MI, H), lambda b, eid: (eid[b], 0, 0)),
            pl.BlockSpec((1, H, MI), lambda b, eid: (eid[b], 0, 0)),
            pl.BlockSpec((1, BT, 1), lambda b, eid: (b, 0, 0)),
        ],
        out_specs=pl.BlockSpec((BT, H), lambda b, eid: (b, 0)),
    ),
    out_shape=jax.ShapeDtypeStruct((PADN, H), jnp.float32),
)


TPW = T // NW


CCH = TPW // 2


def _sc_combine_body(eo_hbm, p0_hbm, p1_hbm, moe_hbm,
                     i0_v, i1_v, r0a, r0b, r1a, r1b,
                     s0a, s0b, s1a, s1b, sw):
    wid = lax.axis_index("s") * NC + lax.axis_index("c")
    base = wid * TPW
    pltpu.sync_copy(p0_hbm.at[pl.ds(base, TPW)], i0_v)
    pltpu.sync_copy(p1_hbm.at[pl.ds(base, TPW)], i1_v)
    c = [
        pltpu.async_copy(eo_hbm.at[i0_v.at[pl.ds(0, CCH)]], r0a, s0a),
        pltpu.async_copy(eo_hbm.at[i0_v.at[pl.ds(CCH, CCH)]], r0b, s0b),
        pltpu.async_copy(eo_hbm.at[i1_v.at[pl.ds(0, CCH)]], r1a, s1a),
        pltpu.async_copy(eo_hbm.at[i1_v.at[pl.ds(CCH, CCH)]], r1b, s1b),
    ]
    writes = []
    for half, (ra, rb) in enumerate(((r0a, r1a), (r0b, r1b))):
        c[half].wait()
        c[half + 2].wait()

        @pl.loop(0, CCH)
        def _(j):
            @pl.loop(0, H // L, unroll=8)
            def _(cchunk):
                sl = pl.ds(cchunk * L, L)
                ra[j, sl] = ra[j, sl] + rb[j, sl]

        writes.append(pltpu.async_copy(
            ra, moe_hbm.at[pl.ds(base + half * CCH, CCH)], sw))
    for w in writes:
        w.wait()


@functools.lru_cache(maxsize=1)
def _build_sc_kernels():
    mesh = plsc.VectorSubcoreMesh(core_axis_name="c", subcore_axis_name="s")
    sc_scatter = functools.partial(
        pl.kernel,
        out_type=[jax.ShapeDtypeStruct((PADN,), jnp.int32),
                  jax.ShapeDtypeStruct((PADN,), jnp.float32)],
        mesh=mesh,
        scratch_types=[
            pltpu.VMEM((T,), jnp.int32), pltpu.VMEM((T,), jnp.int32),
            pltpu.VMEM((T,), jnp.float32), pltpu.VMEM((T,), jnp.float32),
            pltpu.VMEM((PADN,), jnp.int32), pltpu.VMEM((PADN,), jnp.float32),
        ],
        compiler_params=pltpu.CompilerParams(needs_layout_passes=False),
    )(_sc_scatter_body)
    sc_gather = functools.partial(
        pl.kernel,
        out_type=jax.ShapeDtypeStruct((PADN, H2), jnp.int32),
        mesh=mesh,
        scratch_types=[pltpu.VMEM((RPW,), jnp.int32),
                       pltpu.VMEM((RCH, H2), jnp.int32),
                       pltpu.VMEM((RCH, H2), jnp.int32),
                       pltpu.SemaphoreType.DMA, pltpu.SemaphoreType.DMA,
                       pltpu.SemaphoreType.DMA, pltpu.SemaphoreType.DMA],
    )(_sc_gather_body)
    sc_combine = functools.partial(
        pl.kernel,
        out_type=jax.ShapeDtypeStruct((T, H), jnp.float32),
        mesh=mesh,
        scratch_types=[pltpu.VMEM((TPW,), jnp.int32),
                       pltpu.VMEM((TPW,), jnp.int32),
                       pltpu.VMEM((CCH, H), jnp.float32),
                       pltpu.VMEM((CCH, H), jnp.float32),
                       pltpu.VMEM((CCH, H), jnp.float32),
                       pltpu.VMEM((CCH, H), jnp.float32),
                       pltpu.SemaphoreType.DMA, pltpu.SemaphoreType.DMA,
                       pltpu.SemaphoreType.DMA, pltpu.SemaphoreType.DMA,
                       pltpu.SemaphoreType.DMA],
    )(_sc_combine_body)
    return sc_scatter, sc_gather, sc_combine


BTF = 256


def _shared_body(x_ref, wsg_ref, wsu_ref, wsd_ref, wsig_ref, out_ref):
    xf = x_ref[...]
    xb = xf.astype(jnp.bfloat16)
    g = lax.dot_general(xb, wsg_ref[...].astype(jnp.bfloat16),
                        (((1,), (1,)), ((), ())),
                        preferred_element_type=jnp.float32)
    u = lax.dot_general(xb, wsu_ref[...].astype(jnp.bfloat16),
                        (((1,), (1,)), ((), ())),
                        preferred_element_type=jnp.float32)
    h = ((g * jax.nn.sigmoid(g)) * u).astype(jnp.bfloat16)
    shd = lax.dot_general(h, wsd_ref[...].astype(jnp.bfloat16),
                          (((1,), (1,)), ((), ())),
                          preferred_element_type=jnp.float32)
    sg = jax.nn.sigmoid(jnp.sum(xf * wsig_ref[...], axis=1, keepdims=True))
    out_ref[...] = sg * shd


_shared = pl.pallas_call(
    _shared_body,
    grid=(T // BTF,),
    in_specs=[
        pl.BlockSpec((BTF, H), lambda b: (b, 0)),
        pl.BlockSpec((ISH, H), lambda b: (0, 0)),
        pl.BlockSpec((ISH, H), lambda b: (0, 0)),
        pl.BlockSpec((H, ISH), lambda b: (0, 0)),
        pl.BlockSpec((1, H), lambda b: (0, 0)),
    ],
    out_specs=pl.BlockSpec((BTF, H), lambda b: (b, 0)),
    out_shape=jax.ShapeDtypeStruct((T, H), jnp.float32),
)


def _fadd_body(moe_ref, sgsh_ref, out_ref):
    out_ref[...] = moe_ref[...] + sgsh_ref[...]


_fadd = pl.pallas_call(
    _fadd_body,
    grid=(T // 512,),
    in_specs=[
        pl.BlockSpec((512, H), lambda b: (b, 0)),
        pl.BlockSpec((512, H), lambda b: (b, 0)),
    ],
    out_specs=pl.BlockSpec((512, H), lambda b: (b, 0)),
    out_shape=jax.ShapeDtypeStruct((T, H), jnp.float32),
)


def kernel(hidden_states, Wg, We_gate, We_up, We_down,
           Ws_gate, Ws_up, Ws_down, Wsg):
    b, s_, h = hidden_states.shape
    x = hidden_states.reshape(s_, h)
    logits, topw, slots, eid2 = _router(x, Wg)
    p0 = slots[:, 0]
    p1 = slots[:, 1]
    w0 = topw[:, 0]
    w1 = topw[:, 1]
    eid_arr = eid2.reshape(NBLK_PAD)[:NBLK]
    x16 = x.astype(jnp.bfloat16)
    x16i = lax.bitcast_convert_type(
        jnp.stack([x16[:, :H2], x16[:, H2:]], axis=-1), jnp.int32)
    _sc_scatter, _sc_gather, _sc_combine = _build_sc_kernels()
    src_tok, wslot = _sc_scatter(p0, p1, w0, w1)
    sgsh = _shared(x, Ws_gate, Ws_up, Ws_down, Wsg)
    xs32 = _sc_gather(x16i, src_tok)
    eo = _moe_mm(eid_arr, xs32, We_gate, We_up, We_down,
                 wslot.reshape(NBLK, BT, 1))
    moe = _sc_combine(eo, p0, p1)
    final = _fadd(moe, sgsh)
    return final.reshape(b, s_, h), logits


# speedup vs baseline: 1.9960x; 1.2184x over previous
import functools

import jax
import jax.numpy as jnp
from jax import lax
from jax.experimental import pallas as pl
from jax.experimental.pallas import tpu as pltpu
from jax.experimental.pallas import tpu_sc as plsc

T = 2048
H = 768
H2 = H // 2
E = 8
K = 2
MI = 1024
ISH = 2048

BT = 256
NBLK = T * K // BT + E
NBLK_PAD = 32
PADN = NBLK * BT

NC, NS, NW, L = 2, 16, 32, 16
CS = 16
NACT = H2 // CS


def _unpack_bf16(xp):
    lo = lax.bitcast_convert_type(xp << 16, jnp.float32)
    hi = lax.bitcast_convert_type(xp & jnp.int32(-65536), jnp.float32)
    return jnp.concatenate([lo, hi], axis=0).astype(jnp.bfloat16)


def _pack_bf16(r):
    n = r.shape[0] // 2
    r16 = r.astype(jnp.bfloat16).astype(jnp.float32)
    lo = lax.shift_right_logical(
        lax.bitcast_convert_type(r16[:n, :], jnp.int32), 16)
    hi = lax.bitcast_convert_type(r16[n:, :], jnp.int32) & jnp.int32(-65536)
    return lo | hi


def _router_body(x_ref, wg_ref, logits_ref, topw_ref, slots_ref, eid_ref):
    x = x_ref[...]
    wg = wg_ref[...]
    logits = lax.dot_general(x, wg, (((1,), (1,)), ((), ())),
                             preferred_element_type=jnp.float32)
    logits_ref[...] = logits
    m = jnp.max(logits, axis=1, keepdims=True)
    ex = jnp.exp(logits - m)
    rw = ex / jnp.sum(ex, axis=1, keepdims=True)
    iota_e = lax.broadcasted_iota(jnp.int32, (T, E), 1)
    m0 = jnp.max(rw, axis=1, keepdims=True)
    i0 = jnp.min(jnp.where(rw == m0, iota_e, E), axis=1, keepdims=True)
    rw1 = jnp.where(iota_e == i0, -1.0, rw)
    m1 = jnp.max(rw1, axis=1, keepdims=True)
    i1 = jnp.min(jnp.where(rw1 == m1, iota_e, E), axis=1, keepdims=True)
    topw_ref[...] = jnp.concatenate([m0, m1], axis=1)
    oh0 = (iota_e == i0).astype(jnp.float32)
    oh1 = (iota_e == i1).astype(jnp.float32)
    cnt = oh0 + oh1
    s = cnt
    sh = 1
    while sh < T:
        s = s + jnp.concatenate(
            [jnp.zeros((sh, E), jnp.float32), s[:-sh, :]], axis=0)
        sh *= 2
    pre = s - cnt
    tot = s[T - 1:T, :]
    nb = jnp.floor((tot + (BT - 1)) / BT)
    ends = nb
    sh = 1
    while sh < E:
        ends = ends + jnp.concatenate(
            [jnp.zeros((1, sh), jnp.float32), ends[:, :-sh]], axis=1)
        sh *= 2
    offrow = (ends - nb) * float(BT)
    slot0 = jnp.sum(oh0 * (offrow + pre), axis=1, keepdims=True)
    slot1 = jnp.sum(oh1 * (offrow + pre), axis=1, keepdims=True)
    slots_ref[...] = jnp.concatenate([slot0, slot1], axis=1).astype(jnp.int32)
    bio = lax.broadcasted_iota(jnp.int32, (NBLK_PAD, E), 0).astype(jnp.float32)
    ge = (bio >= jnp.broadcast_to(ends, (NBLK_PAD, E))).astype(jnp.float32)
    eidf = jnp.minimum(jnp.sum(ge, axis=1, keepdims=True), float(E - 1))
    eid_ref[...] = eidf.astype(jnp.int32)


_router = pl.pallas_call(
    _router_body,
    out_shape=[
        jax.ShapeDtypeStruct((T, E), jnp.float32),
        jax.ShapeDtypeStruct((T, K), jnp.float32),
        jax.ShapeDtypeStruct((T, K), jnp.int32),
        jax.ShapeDtypeStruct((NBLK_PAD, 1), jnp.int32),
    ],
)


def _sc_scatter_body(p0_hbm, p1_hbm, w0_hbm, w1_hbm, src_hbm, wsl_hbm,
                     p0_v, p1_v, w0_v, w1_v, src_v, wsl_v):
    wid = lax.axis_index("s") * NC + lax.axis_index("c")

    @pl.when(wid == 0)
    def _():
        pltpu.sync_copy(p0_hbm, p0_v)
        pltpu.sync_copy(p1_hbm, p1_v)
        pltpu.sync_copy(w0_hbm, w0_v)
        pltpu.sync_copy(w1_hbm, w1_v)

        @pl.loop(0, PADN // L)
        def _(i):
            src_v[pl.ds(i * L, L)] = jnp.zeros((L,), jnp.int32)
            wsl_v[pl.ds(i * L, L)] = jnp.zeros((L,), jnp.float32)

        @pl.loop(0, T // L)
        def _(i):
            base = i * L
            tvec = lax.iota(jnp.int32, L) + base
            idx0 = p0_v[pl.ds(base, L)]
            idx1 = p1_v[pl.ds(base, L)]
            plsc.store_scatter(src_v, [idx0], tvec)
            plsc.store_scatter(wsl_v, [idx0], w0_v[pl.ds(base, L)])
            plsc.store_scatter(src_v, [idx1], tvec)
            plsc.store_scatter(wsl_v, [idx1], w1_v[pl.ds(base, L)])

        pltpu.sync_copy(src_v, src_hbm)
        pltpu.sync_copy(wsl_v, wsl_hbm)


def _sc_gather_body(xt_hbm, src_hbm, xst_hbm, stage, src_v, outb):
    wid = lax.axis_index("s") * NC + lax.axis_index("c")

    @pl.when(wid < NACT)
    def _():
        pltpu.sync_copy(xt_hbm.at[pl.ds(wid * CS, CS)], stage)
        pltpu.sync_copy(src_hbm, src_v)
        hp = PADN // 2
        for half in range(2):
            @pl.loop(0, hp // L)
            def _(g):
                idxv = src_v[pl.ds(half * hp + g * L, L)]
                for r in range(CS):
                    rv = jnp.full((L,), r, jnp.int32)
                    vals = plsc.load_gather(stage, [rv, idxv])
                    outb[r, pl.ds(g * L, L)] = vals

            pltpu.sync_copy(
                outb, xst_hbm.at[pl.ds(wid * CS, CS), pl.ds(half * hp, hp)])


def _sc_combine_body(eot_hbm, p0_hbm, p1_hbm, moet_hbm,
                     stage, i0_v, i1_v, outb):
    wid = lax.axis_index("s") * NC + lax.axis_index("c")

    @pl.when(wid < NACT)
    def _():
        pltpu.sync_copy(eot_hbm.at[pl.ds(wid * CS, CS)], stage)
        ht = T // 2
        for half in range(2):
            pltpu.sync_copy(p0_hbm.at[pl.ds(half * ht, ht)], i0_v)
            pltpu.sync_copy(p1_hbm.at[pl.ds(half * ht, ht)], i1_v)

            @pl.loop(0, ht // L)
            def _(g):
                i0 = i0_v[pl.ds(g * L, L)]
                i1 = i1_v[pl.ds(g * L, L)]
                for r in range(CS):
                    rv = jnp.full((L,), r, jnp.int32)
                    a = plsc.load_gather(stage, [rv, i0])
                    b = plsc.load_gather(stage, [rv, i1])
                    s = plsc.bitcast(
                        plsc.bitcast(a, jnp.bfloat16)
                        + plsc.bitcast(b, jnp.bfloat16), jnp.int32)
                    outb[r, pl.ds(g * L, L)] = s

            pltpu.sync_copy(
                outb, moet_hbm.at[pl.ds(wid * CS, CS), pl.ds(half * ht, ht)])


@functools.lru_cache(maxsize=1)
def _build_sc_kernels():
    mesh = plsc.VectorSubcoreMesh(core_axis_name="c", subcore_axis_name="s")
    sc_scatter = functools.partial(
        pl.kernel,
        out_type=[jax.ShapeDtypeStruct((PADN,), jnp.int32),
                  jax.ShapeDtypeStruct((PADN,), jnp.float32)],
        mesh=mesh,
        scratch_types=[
            pltpu.VMEM((T,), jnp.int32), pltpu.VMEM((T,), jnp.int32),
            pltpu.VMEM((T,), jnp.float32), pltpu.VMEM((T,), jnp.float32),
            pltpu.VMEM((PADN,), jnp.int32), pltpu.VMEM((PADN,), jnp.float32),
        ],
        compiler_params=pltpu.CompilerParams(needs_layout_passes=False),
    )(_sc_scatter_body)
    sc_gather = functools.partial(
        pl.kernel,
        out_type=jax.ShapeDtypeStruct((H2, PADN), jnp.int32),
        mesh=mesh,
        scratch_types=[pltpu.VMEM((CS, T), jnp.int32),
                       pltpu.VMEM((PADN,), jnp.int32),
                       pltpu.VMEM((CS, PADN // 2), jnp.int32)],
        compiler_params=pltpu.CompilerParams(needs_layout_passes=False),
    )(_sc_gather_body)
    sc_combine = functools.partial(
        pl.kernel,
        out_type=jax.ShapeDtypeStruct((H2, T), jnp.int32),
        mesh=mesh,
        scratch_types=[pltpu.VMEM((CS, PADN), jnp.int32),
                       pltpu.VMEM((T // 2,), jnp.int32),
                       pltpu.VMEM((T // 2,), jnp.int32),
                       pltpu.VMEM((CS, T // 2), jnp.int32)],
        compiler_params=pltpu.CompilerParams(needs_layout_passes=False),
    )(_sc_combine_body)
    return sc_scatter, sc_gather, sc_combine


def _moe_body(eid_ref, xst_ref, weg_ref, weu_ref, wed_ref, wsl_ref, out_ref):
    del eid_ref
    xbt = _unpack_bf16(xst_ref[...])
    wg = weg_ref[0].astype(jnp.bfloat16)
    wu = weu_ref[0].astype(jnp.bfloat16)
    wd = wed_ref[0].astype(jnp.bfloat16)
    gt = lax.dot_general(wg, xbt, (((1,), (0,)), ((), ())),
                         preferred_element_type=jnp.float32)
    ut = lax.dot_general(wu, xbt, (((1,), (0,)), ((), ())),
                         preferred_element_type=jnp.float32)
    ht = ((gt * jax.nn.sigmoid(gt)) * ut).astype(jnp.bfloat16)
    ot = lax.dot_general(wd, ht, (((1,), (0,)), ((), ())),
                         preferred_element_type=jnp.float32)
    out_ref[...] = _pack_bf16(ot * wsl_ref[0])


_moe_mm = pl.pallas_call(
    _moe_body,
    grid_spec=pltpu.PrefetchScalarGridSpec(
        num_scalar_prefetch=1,
        grid=(NBLK,),
        in_specs=[
            pl.BlockSpec((H2, BT), lambda b, eid: (0, b)),
            pl.BlockSpec((1, MI, H), lambda b, eid: (eid[b], 0, 0)),
            pl.BlockSpec((1, MI, H), lambda b, eid: (eid[b], 0, 0)),
            pl.BlockSpec((1, H, MI), lambda b, eid: (eid[b], 0, 0)),
            pl.BlockSpec((1, 1, BT), lambda b, eid: (b, 0, 0)),
        ],
        out_specs=pl.BlockSpec((H2, BT), lambda b, eid: (0, b)),
    ),
    out_shape=jax.ShapeDtypeStruct((H2, PADN), jnp.int32),
)


BTF = 256


def _shared_body(xt_ref, wsg_ref, wsu_ref, wsd_ref, wsig_ref, out_ref):
    xbt = _unpack_bf16(xt_ref[...])
    wsg = wsg_ref[...].astype(jnp.bfloat16)
    wsu = wsu_ref[...].astype(jnp.bfloat16)
    wsd = wsd_ref[...].astype(jnp.bfloat16)
    gt = lax.dot_general(wsg, xbt, (((1,), (0,)), ((), ())),
                         preferred_element_type=jnp.float32)
    ut = lax.dot_general(wsu, xbt, (((1,), (0,)), ((), ())),
                         preferred_element_type=jnp.float32)
    ht = ((gt * jax.nn.sigmoid(gt)) * ut).astype(jnp.bfloat16)
    shdt = lax.dot_general(wsd, ht, (((1,), (0,)), ((), ())),
                           preferred_element_type=jnp.float32)
    sg = jax.nn.sigmoid(jnp.sum(
        xbt.astype(jnp.float32) * wsig_ref[...], axis=0, keepdims=True))
    out_ref[...] = sg * shdt


_shared = pl.pallas_call(
    _shared_body,
    grid=(T // BTF,),
    in_specs=[
        pl.BlockSpec((H2, BTF), lambda b: (0, b)),
        pl.BlockSpec((ISH, H), lambda b: (0, 0)),
        pl.BlockSpec((ISH, H), lambda b: (0, 0)),
        pl.BlockSpec((H, ISH), lambda b: (0, 0)),
        pl.BlockSpec((H, 1), lambda b: (0, 0)),
    ],
    out_specs=pl.BlockSpec((H, BTF), lambda b: (0, b)),
    out_shape=jax.ShapeDtypeStruct((H, T), jnp.float32),
)


def _fadd_body(moet_ref, sgsht_ref, out_ref):
    mp = moet_ref[...]
    lo = lax.bitcast_convert_type(mp << 16, jnp.float32)
    hi = lax.bitcast_convert_type(mp & jnp.int32(-65536), jnp.float32)
    mf = jnp.concatenate([lo, hi], axis=0)
    out_ref[...] = mf + sgsht_ref[...]


_fadd = pl.pallas_call(
    _fadd_body,
    grid=(T // 512,),
    in_specs=[
        pl.BlockSpec((H2, 512), lambda b: (0, b)),
        pl.BlockSpec((H, 512), lambda b: (0, b)),
    ],
    out_specs=pl.BlockSpec((H, 512), lambda b: (0, b)),
    out_shape=jax.ShapeDtypeStruct((H, T), jnp.float32),
)


def kernel(hidden_states, Wg, We_gate, We_up, We_down,
           Ws_gate, Ws_up, Ws_down, Wsg):
    b, s_, h = hidden_states.shape
    x = hidden_states.reshape(s_, h)
    logits, topw, slots, eid2 = _router(x, Wg)
    p0 = slots[:, 0]
    p1 = slots[:, 1]
    w0 = topw[:, 0]
    w1 = topw[:, 1]
    eid_arr = eid2.reshape(NBLK_PAD)[:NBLK]
    x16 = x.astype(jnp.bfloat16)
    xt32 = lax.bitcast_convert_type(
        jnp.stack([x16[:, :H2], x16[:, H2:]], axis=-1), jnp.int32).T
    _sc_scatter, _sc_gather, _sc_combine = _build_sc_kernels()
    src_tok, wslot = _sc_scatter(p0, p1, w0, w1)
    sgsht = _shared(xt32, Ws_gate, Ws_up, Ws_down, Wsg.reshape(H, 1))
    xst32 = _sc_gather(xt32, src_tok)
    eot32 = _moe_mm(eid_arr, xst32, We_gate, We_up, We_down,
                    wslot.reshape(NBLK, 1, BT))
    moet32 = _sc_combine(eot32, p0, p1)
    finalt = _fadd(moet32, sgsht)
    return finalt.T.reshape(b, s_, h), logits


# speedup vs baseline: 2.0035x; 1.0038x over previous
import functools

import jax
import jax.numpy as jnp
from jax import lax
from jax.experimental import pallas as pl
from jax.experimental.pallas import tpu as pltpu
from jax.experimental.pallas import tpu_sc as plsc

T = 2048
H = 768
H2 = H // 2
E = 8
K = 2
MI = 1024
ISH = 2048

BT = 256
NBLK = T * K // BT + E
NBLK_PAD = 32
PADN = NBLK * BT

NC, NS, NW, L = 2, 16, 32, 16
CS = 16
NACT = H2 // CS


def _unpack_bf16(xp):
    lo = lax.bitcast_convert_type(xp << 16, jnp.float32)
    hi = lax.bitcast_convert_type(xp & jnp.int32(-65536), jnp.float32)
    return jnp.concatenate([lo, hi], axis=0).astype(jnp.bfloat16)


def _pack_bf16(r):
    n = r.shape[0] // 2
    r16 = r.astype(jnp.bfloat16).astype(jnp.float32)
    lo = lax.shift_right_logical(
        lax.bitcast_convert_type(r16[:n, :], jnp.int32), 16)
    hi = lax.bitcast_convert_type(r16[n:, :], jnp.int32) & jnp.int32(-65536)
    return lo | hi


def _router_body(x_ref, wg_ref, logits_ref, topw_ref, slots_ref, eid_ref):
    x = x_ref[...]
    wg = wg_ref[...]
    logits = lax.dot_general(x, wg, (((1,), (1,)), ((), ())),
                             preferred_element_type=jnp.float32)
    logits_ref[...] = logits
    m = jnp.max(logits, axis=1, keepdims=True)
    ex = jnp.exp(logits - m)
    rw = ex / jnp.sum(ex, axis=1, keepdims=True)
    iota_e = lax.broadcasted_iota(jnp.int32, (T, E), 1)
    m0 = jnp.max(rw, axis=1, keepdims=True)
    i0 = jnp.min(jnp.where(rw == m0, iota_e, E), axis=1, keepdims=True)
    rw1 = jnp.where(iota_e == i0, -1.0, rw)
    m1 = jnp.max(rw1, axis=1, keepdims=True)
    i1 = jnp.min(jnp.where(rw1 == m1, iota_e, E), axis=1, keepdims=True)
    topw_ref[...] = jnp.concatenate([m0, m1], axis=1)
    oh0 = (iota_e == i0).astype(jnp.float32)
    oh1 = (iota_e == i1).astype(jnp.float32)
    cnt = oh0 + oh1
    s = cnt
    sh = 1
    while sh < T:
        s = s + jnp.concatenate(
            [jnp.zeros((sh, E), jnp.float32), s[:-sh, :]], axis=0)
        sh *= 2
    pre = s - cnt
    tot = s[T - 1:T, :]
    nb = jnp.floor((tot + (BT - 1)) / BT)
    ends = nb
    sh = 1
    while sh < E:
        ends = ends + jnp.concatenate(
            [jnp.zeros((1, sh), jnp.float32), ends[:, :-sh]], axis=1)
        sh *= 2
    offrow = (ends - nb) * float(BT)
    slot0 = jnp.sum(oh0 * (offrow + pre), axis=1, keepdims=True)
    slot1 = jnp.sum(oh1 * (offrow + pre), axis=1, keepdims=True)
    slots_ref[...] = jnp.concatenate([slot0, slot1], axis=1).astype(jnp.int32)
    bio = lax.broadcasted_iota(jnp.int32, (NBLK_PAD, E), 0).astype(jnp.float32)
    ge = (bio >= jnp.broadcast_to(ends, (NBLK_PAD, E))).astype(jnp.float32)
    eidf = jnp.minimum(jnp.sum(ge, axis=1, keepdims=True), float(E - 1))
    eid_ref[...] = eidf.astype(jnp.int32)


_router = pl.pallas_call(
    _router_body,
    out_shape=[
        jax.ShapeDtypeStruct((T, E), jnp.float32),
        jax.ShapeDtypeStruct((T, K), jnp.float32),
        jax.ShapeDtypeStruct((T, K), jnp.int32),
        jax.ShapeDtypeStruct((NBLK_PAD, 1), jnp.int32),
    ],
)


def _sc_scatter_body(p0_hbm, p1_hbm, w0_hbm, w1_hbm, src_hbm, wsl_hbm,
                     p0_v, p1_v, w0_v, w1_v, src_v, wsl_v):
    wid = lax.axis_index("s") * NC + lax.axis_index("c")

    @pl.when(wid == 0)
    def _():
        pltpu.sync_copy(p0_hbm, p0_v)
        pltpu.sync_copy(p1_hbm, p1_v)
        pltpu.sync_copy(w0_hbm, w0_v)
        pltpu.sync_copy(w1_hbm, w1_v)

        @pl.loop(0, PADN // L)
        def _(i):
            src_v[pl.ds(i * L, L)] = jnp.zeros((L,), jnp.int32)
            wsl_v[pl.ds(i * L, L)] = jnp.zeros((L,), jnp.float32)

        @pl.loop(0, T // L)
        def _(i):
            base = i * L
            tvec = lax.iota(jnp.int32, L) + base
            idx0 = p0_v[pl.ds(base, L)]
            idx1 = p1_v[pl.ds(base, L)]
            plsc.store_scatter(src_v, [idx0], tvec)
            plsc.store_scatter(wsl_v, [idx0], w0_v[pl.ds(base, L)])
            plsc.store_scatter(src_v, [idx1], tvec)
            plsc.store_scatter(wsl_v, [idx1], w1_v[pl.ds(base, L)])

        pltpu.sync_copy(src_v, src_hbm)
        pltpu.sync_copy(wsl_v, wsl_hbm)


def _sc_gather_body(xt_hbm, src_hbm, xst_hbm, stage, src_v, outb):
    wid = lax.axis_index("s") * NC + lax.axis_index("c")

    @pl.when(wid < NACT)
    def _():
        pltpu.sync_copy(xt_hbm.at[pl.ds(wid * CS, CS)], stage)
        pltpu.sync_copy(src_hbm, src_v)
        hp = PADN // 2
        for half in range(2):
            @pl.loop(0, hp // L, unroll=4)
            def _(g):
                idxv = src_v[pl.ds(half * hp + g * L, L)]
                for r in range(CS):
                    rv = jnp.full((L,), r, jnp.int32)
                    vals = plsc.load_gather(stage, [rv, idxv])
                    outb[r, pl.ds(g * L, L)] = vals

            pltpu.sync_copy(
                outb, xst_hbm.at[pl.ds(wid * CS, CS), pl.ds(half * hp, hp)])


def _sc_combine_body(eot_hbm, p0_hbm, p1_hbm, moet_hbm,
                     stage, i0_v, i1_v, outb):
    wid = lax.axis_index("s") * NC + lax.axis_index("c")

    @pl.when(wid < NACT)
    def _():
        pltpu.sync_copy(eot_hbm.at[pl.ds(wid * CS, CS)], stage)
        ht = T // 2
        for half in range(2):
            pltpu.sync_copy(p0_hbm.at[pl.ds(half * ht, ht)], i0_v)
            pltpu.sync_copy(p1_hbm.at[pl.ds(half * ht, ht)], i1_v)

            @pl.loop(0, ht // L, unroll=4)
            def _(g):
                i0 = i0_v[pl.ds(g * L, L)]
                i1 = i1_v[pl.ds(g * L, L)]
                for r in range(CS):
                    rv = jnp.full((L,), r, jnp.int32)
                    a = plsc.load_gather(stage, [rv, i0])
                    b = plsc.load_gather(stage, [rv, i1])
                    s = plsc.bitcast(
                        plsc.bitcast(a, jnp.bfloat16)
                        + plsc.bitcast(b, jnp.bfloat16), jnp.int32)
                    outb[r, pl.ds(g * L, L)] = s

            pltpu.sync_copy(
                outb, moet_hbm.at[pl.ds(wid * CS, CS), pl.ds(half * ht, ht)])


@functools.lru_cache(maxsize=1)
def _build_sc_kernels():
    mesh = plsc.VectorSubcoreMesh(core_axis_name="c", subcore_axis_name="s")
    sc_scatter = functools.partial(
        pl.kernel,
        out_type=[jax.ShapeDtypeStruct((PADN,), jnp.int32),
                  jax.ShapeDtypeStruct((PADN,), jnp.float32)],
        mesh=mesh,
        scratch_types=[
            pltpu.VMEM((T,), jnp.int32), pltpu.VMEM((T,), jnp.int32),
            pltpu.VMEM((T,), jnp.float32), pltpu.VMEM((T,), jnp.float32),
            pltpu.VMEM((PADN,), jnp.int32), pltpu.VMEM((PADN,), jnp.float32),
        ],
        compiler_params=pltpu.CompilerParams(needs_layout_passes=False),
    )(_sc_scatter_body)
    sc_gather = functools.partial(
        pl.kernel,
        out_type=jax.ShapeDtypeStruct((H2, PADN), jnp.int32),
        mesh=mesh,
        scratch_types=[pltpu.VMEM((CS, T), jnp.int32),
                       pltpu.VMEM((PADN,), jnp.int32),
                       pltpu.VMEM((CS, PADN // 2), jnp.int32)],
        compiler_params=pltpu.CompilerParams(needs_layout_passes=False),
    )(_sc_gather_body)
    sc_combine = functools.partial(
        pl.kernel,
        out_type=jax.ShapeDtypeStruct((H2, T), jnp.int32),
        mesh=mesh,
        scratch_types=[pltpu.VMEM((CS, PADN), jnp.int32),
                       pltpu.VMEM((T // 2,), jnp.int32),
                       pltpu.VMEM((T // 2,), jnp.int32),
                       pltpu.VMEM((CS, T // 2), jnp.int32)],
        compiler_params=pltpu.CompilerParams(needs_layout_passes=False),
    )(_sc_combine_body)
    return sc_scatter, sc_gather, sc_combine


def _moe_body(eid_ref, xst_ref, weg_ref, weu_ref, wed_ref, wsl_ref, out_ref):
    del eid_ref
    xbt = _unpack_bf16(xst_ref[...])
    wg = weg_ref[0].astype(jnp.bfloat16)
    wu = weu_ref[0].astype(jnp.bfloat16)
    wd = wed_ref[0].astype(jnp.bfloat16)
    gt = lax.dot_general(wg, xbt, (((1,), (0,)), ((), ())),
                         preferred_element_type=jnp.float32)
    ut = lax.dot_general(wu, xbt, (((1,), (0,)), ((), ())),
                         preferred_element_type=jnp.float32)
    ht = ((gt * jax.nn.sigmoid(gt)) * ut).astype(jnp.bfloat16)
    ot = lax.dot_general(wd, ht, (((1,), (0,)), ((), ())),
                         preferred_element_type=jnp.float32)
    out_ref[...] = _pack_bf16(ot * wsl_ref[0])


_moe_mm = pl.pallas_call(
    _moe_body,
    grid_spec=pltpu.PrefetchScalarGridSpec(
        num_scalar_prefetch=1,
        grid=(NBLK,),
        in_specs=[
            pl.BlockSpec((H2, BT), lambda b, eid: (0, b)),
            pl.BlockSpec((1, MI, H), lambda b, eid: (eid[b], 0, 0)),
            pl.BlockSpec((1, MI, H), lambda b, eid: (eid[b], 0, 0)),
            pl.BlockSpec((1, H, MI), lambda b, eid: (eid[b], 0, 0)),
            pl.BlockSpec((1, 1, BT), lambda b, eid: (b, 0, 0)),
        ],
        out_specs=pl.BlockSpec((H2, BT), lambda b, eid: (0, b)),
    ),
    out_shape=jax.ShapeDtypeStruct((H2, PADN), jnp.int32),
)


BTF = 256


def _shared_body(xt_ref, wsg_ref, wsu_ref, wsd_ref, wsig_ref, out_ref):
    xbt = _unpack_bf16(xt_ref[...])
    wsg = wsg_ref[...].astype(jnp.bfloat16)
    wsu = wsu_ref[...].astype(jnp.bfloat16)
    wsd = wsd_ref[...].astype(jnp.bfloat16)
    gt = lax.dot_general(wsg, xbt, (((1,), (0,)), ((), ())),
                         preferred_element_type=jnp.float32)
    ut = lax.dot_general(wsu, xbt, (((1,), (0,)), ((), ())),
                         preferred_element_type=jnp.float32)
    ht = ((gt * jax.nn.sigmoid(gt)) * ut).astype(jnp.bfloat16)
    shdt = lax.dot_general(wsd, ht, (((1,), (0,)), ((), ())),
                           preferred_element_type=jnp.float32)
    sg = jax.nn.sigmoid(jnp.sum(
        xbt.astype(jnp.float32) * wsig_ref[...], axis=0, keepdims=True))
    out_ref[...] = sg * shdt


_shared = pl.pallas_call(
    _shared_body,
    grid=(T // BTF,),
    in_specs=[
        pl.BlockSpec((H2, BTF), lambda b: (0, b)),
        pl.BlockSpec((ISH, H), lambda b: (0, 0)),
        pl.BlockSpec((ISH, H), lambda b: (0, 0)),
        pl.BlockSpec((H, ISH), lambda b: (0, 0)),
        pl.BlockSpec((H, 1), lambda b: (0, 0)),
    ],
    out_specs=pl.BlockSpec((H, BTF), lambda b: (0, b)),
    out_shape=jax.ShapeDtypeStruct((H, T), jnp.float32),
)


def _fadd_body(moet_ref, sgsht_ref, out_ref):
    mp = moet_ref[...]
    lo = lax.bitcast_convert_type(mp << 16, jnp.float32)
    hi = lax.bitcast_convert_type(mp & jnp.int32(-65536), jnp.float32)
    mf = jnp.concatenate([lo, hi], axis=0)
    out_ref[...] = mf + sgsht_ref[...]


_fadd = pl.pallas_call(
    _fadd_body,
    grid=(T // 512,),
    in_specs=[
        pl.BlockSpec((H2, 512), lambda b: (0, b)),
        pl.BlockSpec((H, 512), lambda b: (0, b)),
    ],
    out_specs=pl.BlockSpec((H, 512), lambda b: (0, b)),
    out_shape=jax.ShapeDtypeStruct((H, T), jnp.float32),
)


def kernel(hidden_states, Wg, We_gate, We_up, We_down,
           Ws_gate, Ws_up, Ws_down, Wsg):
    b, s_, h = hidden_states.shape
    x = hidden_states.reshape(s_, h)
    logits, topw, slots, eid2 = _router(x, Wg)
    p0 = slots[:, 0]
    p1 = slots[:, 1]
    w0 = topw[:, 0]
    w1 = topw[:, 1]
    eid_arr = eid2.reshape(NBLK_PAD)[:NBLK]
    x16 = x.astype(jnp.bfloat16)
    xt32 = lax.bitcast_convert_type(
        jnp.stack([x16[:, :H2], x16[:, H2:]], axis=-1), jnp.int32).T
    _sc_scatter, _sc_gather, _sc_combine = _build_sc_kernels()
    src_tok, wslot = _sc_scatter(p0, p1, w0, w1)
    sgsht = _shared(xt32, Ws_gate, Ws_up, Ws_down, Wsg.reshape(H, 1))
    xst32 = _sc_gather(xt32, src_tok)
    eot32 = _moe_mm(eid_arr, xst32, We_gate, We_up, We_down,
                    wslot.reshape(NBLK, 1, BT))
    moet32 = _sc_combine(eot32, p0, p1)
    finalt = _fadd(moet32, sgsht)
    return finalt.T.reshape(b, s_, h), logits


# speedup vs baseline: 2.0702x; 1.0333x over previous
import functools

import jax
import jax.numpy as jnp
from jax import lax
from jax.experimental import pallas as pl
from jax.experimental.pallas import tpu as pltpu
from jax.experimental.pallas import tpu_sc as plsc

T = 2048
H = 768
H2 = H // 2
E = 8
K = 2
MI = 1024
ISH = 2048

BT = 256
NBLK = T * K // BT + E
NBLK_PAD = 32
PADN = NBLK * BT

NC, NS, NW, L = 2, 16, 32, 16
CS = 16
NACT = H2 // CS


def _unpack_bf16(xp):
    lo = lax.bitcast_convert_type(xp << 16, jnp.float32)
    hi = lax.bitcast_convert_type(xp & jnp.int32(-65536), jnp.float32)
    return jnp.concatenate([lo, hi], axis=0).astype(jnp.bfloat16)


def _pack_bf16(r):
    n = r.shape[0] // 2
    r16 = r.astype(jnp.bfloat16).astype(jnp.float32)
    lo = lax.shift_right_logical(
        lax.bitcast_convert_type(r16[:n, :], jnp.int32), 16)
    hi = lax.bitcast_convert_type(r16[n:, :], jnp.int32) & jnp.int32(-65536)
    return lo | hi


def _router_body(x_ref, wg_ref, logits_ref, topw_ref, slots_ref, eid_ref,
                 xt_ref):
    x = x_ref[...]
    wg = wg_ref[...]
    logits = lax.dot_general(x, wg, (((1,), (1,)), ((), ())),
                             preferred_element_type=jnp.float32)
    logits_ref[...] = logits
    m = jnp.max(logits, axis=1, keepdims=True)
    ex = jnp.exp(logits - m)
    rw = ex / jnp.sum(ex, axis=1, keepdims=True)
    iota_e = lax.broadcasted_iota(jnp.int32, (T, E), 1)
    m0 = jnp.max(rw, axis=1, keepdims=True)
    i0 = jnp.min(jnp.where(rw == m0, iota_e, E), axis=1, keepdims=True)
    rw1 = jnp.where(iota_e == i0, -1.0, rw)
    m1 = jnp.max(rw1, axis=1, keepdims=True)
    i1 = jnp.min(jnp.where(rw1 == m1, iota_e, E), axis=1, keepdims=True)
    topw_ref[...] = jnp.concatenate([m0, m1], axis=1)
    oh0 = (iota_e == i0).astype(jnp.float32)
    oh1 = (iota_e == i1).astype(jnp.float32)
    cnt = oh0 + oh1
    s = cnt
    sh = 1
    while sh < T:
        s = s + jnp.concatenate(
            [jnp.zeros((sh, E), jnp.float32), s[:-sh, :]], axis=0)
        sh *= 2
    pre = s - cnt
    tot = s[T - 1:T, :]
    nb = jnp.floor((tot + (BT - 1)) / BT)
    ends = nb
    sh = 1
    while sh < E:
        ends = ends + jnp.concatenate(
            [jnp.zeros((1, sh), jnp.float32), ends[:, :-sh]], axis=1)
        sh *= 2
    offrow = (ends - nb) * float(BT)
    slot0 = jnp.sum(oh0 * (offrow + pre), axis=1, keepdims=True)
    slot1 = jnp.sum(oh1 * (offrow + pre), axis=1, keepdims=True)
    slots_ref[...] = jnp.concatenate([slot0, slot1], axis=1).astype(jnp.int32)
    bio = lax.broadcasted_iota(jnp.int32, (NBLK_PAD, E), 0).astype(jnp.float32)
    ge = (bio >= jnp.broadcast_to(ends, (NBLK_PAD, E))).astype(jnp.float32)
    eidf = jnp.minimum(jnp.sum(ge, axis=1, keepdims=True), float(E - 1))
    eid_ref[...] = eidf.astype(jnp.int32)
    xr = x.astype(jnp.bfloat16).astype(jnp.float32)
    lo = lax.shift_right_logical(
        lax.bitcast_convert_type(xr[:, :H2], jnp.int32), 16)
    hi = lax.bitcast_convert_type(xr[:, H2:], jnp.int32) & jnp.int32(-65536)
    xt_ref[...] = (lo | hi).T


_router = pl.pallas_call(
    _router_body,
    out_shape=[
        jax.ShapeDtypeStruct((T, E), jnp.float32),
        jax.ShapeDtypeStruct((T, K), jnp.float32),
        jax.ShapeDtypeStruct((T, K), jnp.int32),
        jax.ShapeDtypeStruct((NBLK_PAD, 1), jnp.int32),
        jax.ShapeDtypeStruct((H2, T), jnp.int32),
    ],
)


def _sc_scatter_body(p0_hbm, p1_hbm, w0_hbm, w1_hbm, src_hbm, wsl_hbm,
                     p0_v, p1_v, w0_v, w1_v, src_v, wsl_v):
    wid = lax.axis_index("s") * NC + lax.axis_index("c")

    @pl.when(wid == 0)
    def _():
        pltpu.sync_copy(p0_hbm, p0_v)
        pltpu.sync_copy(p1_hbm, p1_v)
        pltpu.sync_copy(w0_hbm, w0_v)
        pltpu.sync_copy(w1_hbm, w1_v)

        @pl.loop(0, PADN // L)
        def _(i):
            src_v[pl.ds(i * L, L)] = jnp.zeros((L,), jnp.int32)
            wsl_v[pl.ds(i * L, L)] = jnp.zeros((L,), jnp.float32)

        @pl.loop(0, T // L)
        def _(i):
            base = i * L
            tvec = lax.iota(jnp.int32, L) + base
            idx0 = p0_v[pl.ds(base, L)]
            idx1 = p1_v[pl.ds(base, L)]
            plsc.store_scatter(src_v, [idx0], tvec)
            plsc.store_scatter(wsl_v, [idx0], w0_v[pl.ds(base, L)])
            plsc.store_scatter(src_v, [idx1], tvec)
            plsc.store_scatter(wsl_v, [idx1], w1_v[pl.ds(base, L)])

        pltpu.sync_copy(src_v, src_hbm)
        pltpu.sync_copy(wsl_v, wsl_hbm)


def _sc_gather_body(xt_hbm, src_hbm, xst_hbm, stage, src_v, outb):
    wid = lax.axis_index("s") * NC + lax.axis_index("c")

    @pl.when(wid < NACT)
    def _():
        pltpu.sync_copy(xt_hbm.at[pl.ds(wid * CS, CS)], stage)
        pltpu.sync_copy(src_hbm, src_v)
        hp = PADN // 2
        for half in range(2):
            @pl.loop(0, hp // L, unroll=4)
            def _(g):
                idxv = src_v[pl.ds(half * hp + g * L, L)]
                for r in range(CS):
                    rv = jnp.full((L,), r, jnp.int32)
                    vals = plsc.load_gather(stage, [rv, idxv])
                    outb[r, pl.ds(g * L, L)] = vals

            pltpu.sync_copy(
                outb, xst_hbm.at[pl.ds(wid * CS, CS), pl.ds(half * hp, hp)])


def _sc_combine_body(eot_hbm, p0_hbm, p1_hbm, moet_hbm,
                     stage, i0_v, i1_v, outb):
    wid = lax.axis_index("s") * NC + lax.axis_index("c")

    @pl.when(wid < NACT)
    def _():
        pltpu.sync_copy(eot_hbm.at[pl.ds(wid * CS, CS)], stage)
        ht = T // 2
        for half in range(2):
            pltpu.sync_copy(p0_hbm.at[pl.ds(half * ht, ht)], i0_v)
            pltpu.sync_copy(p1_hbm.at[pl.ds(half * ht, ht)], i1_v)

            @pl.loop(0, ht // L, unroll=4)
            def _(g):
                i0 = i0_v[pl.ds(g * L, L)]
                i1 = i1_v[pl.ds(g * L, L)]
                for r in range(CS):
                    rv = jnp.full((L,), r, jnp.int32)
                    a = plsc.load_gather(stage, [rv, i0])
                    b = plsc.load_gather(stage, [rv, i1])
                    s = plsc.bitcast(
                        plsc.bitcast(a, jnp.bfloat16)
                        + plsc.bitcast(b, jnp.bfloat16), jnp.int32)
                    outb[r, pl.ds(g * L, L)] = s

            pltpu.sync_copy(
                outb, moet_hbm.at[pl.ds(wid * CS, CS), pl.ds(half * ht, ht)])


@functools.lru_cache(maxsize=1)
def _build_sc_kernels():
    mesh = plsc.VectorSubcoreMesh(core_axis_name="c", subcore_axis_name="s")
    sc_scatter = functools.partial(
        pl.kernel,
        out_type=[jax.ShapeDtypeStruct((PADN,), jnp.int32),
                  jax.ShapeDtypeStruct((PADN,), jnp.float32)],
        mesh=mesh,
        scratch_types=[
            pltpu.VMEM((T,), jnp.int32), pltpu.VMEM((T,), jnp.int32),
            pltpu.VMEM((T,), jnp.float32), pltpu.VMEM((T,), jnp.float32),
            pltpu.VMEM((PADN,), jnp.int32), pltpu.VMEM((PADN,), jnp.float32),
        ],
        compiler_params=pltpu.CompilerParams(needs_layout_passes=False),
    )(_sc_scatter_body)
    sc_gather = functools.partial(
        pl.kernel,
        out_type=jax.ShapeDtypeStruct((H2, PADN), jnp.int32),
        mesh=mesh,
        scratch_types=[pltpu.VMEM((CS, T), jnp.int32),
                       pltpu.VMEM((PADN,), jnp.int32),
                       pltpu.VMEM((CS, PADN // 2), jnp.int32)],
        compiler_params=pltpu.CompilerParams(needs_layout_passes=False),
    )(_sc_gather_body)
    sc_combine = functools.partial(
        pl.kernel,
        out_type=jax.ShapeDtypeStruct((H2, T), jnp.int32),
        mesh=mesh,
        scratch_types=[pltpu.VMEM((CS, PADN), jnp.int32),
                       pltpu.VMEM((T // 2,), jnp.int32),
                       pltpu.VMEM((T // 2,), jnp.int32),
                       pltpu.VMEM((CS, T // 2), jnp.int32)],
        compiler_params=pltpu.CompilerParams(needs_layout_passes=False),
    )(_sc_combine_body)
    return sc_scatter, sc_gather, sc_combine


def _moe_body(eid_ref, xst_ref, weg_ref, weu_ref, wed_ref, wsl_ref, out_ref):
    del eid_ref
    xbt = _unpack_bf16(xst_ref[...])
    wg = weg_ref[0].astype(jnp.bfloat16)
    wu = weu_ref[0].astype(jnp.bfloat16)
    wd = wed_ref[0].astype(jnp.bfloat16)
    gt = lax.dot_general(wg, xbt, (((1,), (0,)), ((), ())),
                         preferred_element_type=jnp.float32)
    ut = lax.dot_general(wu, xbt, (((1,), (0,)), ((), ())),
                         preferred_element_type=jnp.float32)
    ht = ((gt * jax.nn.sigmoid(gt)) * ut).astype(jnp.bfloat16)
    ot = lax.dot_general(wd, ht, (((1,), (0,)), ((), ())),
                         preferred_element_type=jnp.float32)
    out_ref[...] = _pack_bf16(ot * wsl_ref[0])


_moe_mm = pl.pallas_call(
    _moe_body,
    grid_spec=pltpu.PrefetchScalarGridSpec(
        num_scalar_prefetch=1,
        grid=(NBLK,),
        in_specs=[
            pl.BlockSpec((H2, BT), lambda b, eid: (0, b)),
            pl.BlockSpec((1, MI, H), lambda b, eid: (eid[b], 0, 0)),
            pl.BlockSpec((1, MI, H), lambda b, eid: (eid[b], 0, 0)),
            pl.BlockSpec((1, H, MI), lambda b, eid: (eid[b], 0, 0)),
            pl.BlockSpec((1, 1, BT), lambda b, eid: (b, 0, 0)),
        ],
        out_specs=pl.BlockSpec((H2, BT), lambda b, eid: (0, b)),
    ),
    out_shape=jax.ShapeDtypeStruct((H2, PADN), jnp.int32),
)


BTF = 256


def _shared_body(xt_ref, wsg_ref, wsu_ref, wsd_ref, wsig_ref, out_ref):
    xbt = _unpack_bf16(xt_ref[...])
    wsg = wsg_ref[...].astype(jnp.bfloat16)
    wsu = wsu_ref[...].astype(jnp.bfloat16)
    wsd = wsd_ref[...].astype(jnp.bfloat16)
    gt = lax.dot_general(wsg, xbt, (((1,), (0,)), ((), ())),
                         preferred_element_type=jnp.float32)
    ut = lax.dot_general(wsu, xbt, (((1,), (0,)), ((), ())),
                         preferred_element_type=jnp.float32)
    ht = ((gt * jax.nn.sigmoid(gt)) * ut).astype(jnp.bfloat16)
    shdt = lax.dot_general(wsd, ht, (((1,), (0,)), ((), ())),
                           preferred_element_type=jnp.float32)
    sg = jax.nn.sigmoid(jnp.sum(
        xbt.astype(jnp.float32) * wsig_ref[...], axis=0, keepdims=True))
    out_ref[...] = sg * shdt


_shared = pl.pallas_call(
    _shared_body,
    grid=(T // BTF,),
    in_specs=[
        pl.BlockSpec((H2, BTF), lambda b: (0, b)),
        pl.BlockSpec((ISH, H), lambda b: (0, 0)),
        pl.BlockSpec((ISH, H), lambda b: (0, 0)),
        pl.BlockSpec((H, ISH), lambda b: (0, 0)),
        pl.BlockSpec((H, 1), lambda b: (0, 0)),
    ],
    out_specs=pl.BlockSpec((H, BTF), lambda b: (0, b)),
    out_shape=jax.ShapeDtypeStruct((H, T), jnp.float32),
)


def _fadd_body(moet_ref, sgsht_ref, out_ref):
    mp = moet_ref[...]
    lo = lax.bitcast_convert_type(mp << 16, jnp.float32)
    hi = lax.bitcast_convert_type(mp & jnp.int32(-65536), jnp.float32)
    mf = jnp.concatenate([lo, hi], axis=0)
    out_ref[...] = (mf + sgsht_ref[...]).T


_fadd = pl.pallas_call(
    _fadd_body,
    grid=(T // 512,),
    in_specs=[
        pl.BlockSpec((H2, 512), lambda b: (0, b)),
        pl.BlockSpec((H, 512), lambda b: (0, b)),
    ],
    out_specs=pl.BlockSpec((512, H), lambda b: (b, 0)),
    out_shape=jax.ShapeDtypeStruct((T, H), jnp.float32),
)


def kernel(hidden_states, Wg, We_gate, We_up, We_down,
           Ws_gate, Ws_up, Ws_down, Wsg):
    b, s_, h = hidden_states.shape
    x = hidden_states.reshape(s_, h)
    logits, topw, slots, eid2, xt32 = _router(x, Wg)
    p0 = slots[:, 0]
    p1 = slots[:, 1]
    w0 = topw[:, 0]
    w1 = topw[:, 1]
    eid_arr = eid2.reshape(NBLK_PAD)[:NBLK]
    _sc_scatter, _sc_gather, _sc_combine = _build_sc_kernels()
    src_tok, wslot = _sc_scatter(p0, p1, w0, w1)
    sgsht = _shared(xt32, Ws_gate, Ws_up, Ws_down, Wsg.reshape(H, 1))
    xst32 = _sc_gather(xt32, src_tok)
    eot32 = _moe_mm(eid_arr, xst32, We_gate, We_up, We_down,
                    wslot.reshape(NBLK, 1, BT))
    moet32 = _sc_combine(eot32, p0, p1)
    final = _fadd(moet32, sgsht)
    return final.reshape(b, s_, h), logits
